# baseline (speedup 1.0000x reference)
import functools

import jax
import jax.numpy as jnp
from jax import lax
from jax.experimental import pallas as pl
from jax.experimental.pallas import tpu as pltpu
from jax.experimental.pallas import tpu_sc as plsc

_V, _D, _B = 1000000, 64, 16384
_C = 128
_HALF_PAIRS = _V // 4
_IGN = -1


@functools.lru_cache(maxsize=None)
def _build_gather():
    info = plsc.get_sparse_core_info()
    nc, ns = info.num_cores, info.num_subcores
    nw = nc * ns
    b_per_w = _B // nw
    nchunk = b_per_w // _C
    mesh = plsc.VectorSubcoreMesh(core_axis_name="c", subcore_axis_name="s")

    @functools.partial(
        pl.kernel,
        mesh=mesh,
        out_type=jax.ShapeDtypeStruct((_B, _D), jnp.float32),
        scratch_types=[
            pltpu.VMEM((b_per_w,), jnp.int32),
            pltpu.VMEM((b_per_w,), jnp.int32),
            pltpu.VMEM((b_per_w,), jnp.int32),
            pltpu.VMEM((_C, 2 * _D), jnp.float32),
            pltpu.VMEM((_C, 2 * _D), jnp.float32),
            pltpu.VMEM((b_per_w, _D), jnp.float32),
            pltpu.SemaphoreType.DMA,
        ],
        compiler_params=pltpu.CompilerParams(disable_bounds_checks=True),
    )
    def gather_kernel(
        tlo, thi, idx_hbm, out_hbm, idx_v, p_lo, p_hi, g0, g1, rows_v, sem
    ):
        wid = lax.axis_index("s") * nc + lax.axis_index("c")
        base = wid * b_per_w
        gaths = (g0, g1)

        pltpu.sync_copy(idx_hbm.at[pl.ds(base, b_per_w)], idx_v)

        def mask_body(g, carry):
            vec = idx_v[pl.ds(g * 16, 16)]
            pair = vec >> 1
            is_lo = pair < _HALF_PAIRS
            p_lo[pl.ds(g * 16, 16)] = jnp.where(is_lo, pair, _IGN)
            p_hi[pl.ds(g * 16, 16)] = jnp.where(is_lo, _IGN, pair - _HALF_PAIRS)
            return carry

        lax.fori_loop(0, b_per_w // 16, mask_body, 0)

        def start_chunk(c):
            pltpu.async_copy(
                tlo.at[plsc.Indices(p_lo.at[pl.ds(c * _C, _C)], ignored_value=_IGN)],
                gaths[c % 2],
                sem,
            )
            pltpu.async_copy(
                thi.at[plsc.Indices(p_hi.at[pl.ds(c * _C, _C)], ignored_value=_IGN)],
                gaths[c % 2],
                sem,
            )

        def wait_chunk(c):
            for _ in range(2):
                pltpu.make_async_copy(
                    tlo.at[plsc.Indices(p_lo.at[pl.ds(c * _C, _C)], ignored_value=_IGN)],
                    gaths[c % 2],
                    sem,
                ).wait()

        start_chunk(0)
        for c in range(nchunk):
            if c + 1 < nchunk:
                start_chunk(c + 1)
            wait_chunk(c)
            gath = gaths[c % 2]

            def ext_body(g, carry):
                vec = idx_v[pl.ds(c * _C + g * 16, 16)]
                for j in range(16):
                    r = vec[j]
                    off = (r & 1) * _D
                    row = c * _C + g * 16 + j
                    i = g * 16 + j
                    for k in range(_D // 16):
                        rows_v[row, pl.ds(k * 16, 16)] = gath[i, pl.ds(off + k * 16, 16)]
                return carry

            lax.fori_loop(0, _C // 16, ext_body, 0)

        pltpu.sync_copy(rows_v, out_hbm.at[pl.ds(base, b_per_w)])

    return gather_kernel


def kernel(inputs, index):
    tlo = inputs[: _V // 2].reshape(_HALF_PAIRS, 2 * _D)
    thi = inputs[_V // 2 :].reshape(_HALF_PAIRS, 2 * _D)
    out = _build_gather()(tlo, thi, index.astype(jnp.int32))
    return out[None]


# speedup vs baseline: 2.5006x; 2.5006x over previous
import functools

import jax
import jax.numpy as jnp
from jax import lax
from jax.experimental import pallas as pl
from jax.experimental.pallas import tpu as pltpu
from jax.experimental.pallas import tpu_sc as plsc

_V, _D, _B = 1000000, 64, 16384


@functools.lru_cache(maxsize=None)
def _build_gather():
    info = plsc.get_sparse_core_info()
    nc, ns = info.num_cores, info.num_subcores
    nw = nc * ns
    b_per_w = _B // nw
    mesh = plsc.VectorSubcoreMesh(core_axis_name="c", subcore_axis_name="s")

    @functools.partial(
        pl.kernel,
        mesh=mesh,
        out_type=jax.ShapeDtypeStruct((_B, _D), jnp.float32),
        scratch_types=[
            pltpu.VMEM((b_per_w,), jnp.int32),
            pltpu.VMEM((b_per_w, _D), jnp.float32),
            pltpu.SemaphoreType.DMA,
            pltpu.SemaphoreType.DMA,
        ],
        compiler_params=pltpu.CompilerParams(disable_bounds_checks=True),
    )
    def gather_kernel(table_hbm, idx_hbm, out_hbm, idx_v, rows_v, sem, sem_out):
        wid = lax.axis_index("s") * nc + lax.axis_index("c")
        base = wid * b_per_w
        ngroups = b_per_w // 16

        pltpu.sync_copy(idx_hbm.at[pl.ds(base, b_per_w)], idx_v)

        def fire_group(g):
            vec = idx_v[pl.ds(g * 16, 16)]
            for j in range(16):
                r = vec[j]
                pltpu.make_async_copy(
                    table_hbm.at[pl.ds(r, 1)],
                    rows_v.at[pl.ds(g * 16 + j, 1)],
                    sem,
                ).start()

        def drain_group(g):
            pltpu.make_async_copy(
                table_hbm.at[pl.ds(0, 16)], rows_v.at[pl.ds(0, 16)], sem
            ).wait()
            pltpu.make_async_copy(
                rows_v.at[pl.ds(g * 16, 16)],
                out_hbm.at[pl.ds(base + g * 16, 16)],
                sem_out,
            ).start()

        def prime_body(g, carry):
            fire_group(g)
            return carry

        def main_body(g, carry):
            fire_group(g)
            drain_group(g - nprime)
            return carry

        def tail_body(g, carry):
            drain_group(ngroups - nprime + g)
            return carry

        def out_wait_body(g, carry):
            pltpu.make_async_copy(
                rows_v.at[pl.ds(0, 16)], out_hbm.at[pl.ds(0, 16)], sem_out
            ).wait()
            return carry

        nprime = 4
        lax.fori_loop(0, nprime, prime_body, 0)
        lax.fori_loop(nprime, ngroups, main_body, 0)
        lax.fori_loop(0, nprime, tail_body, 0)
        lax.fori_loop(0, ngroups, out_wait_body, 0)

    return gather_kernel


def kernel(inputs, index):
    out = _build_gather()(inputs, index.astype(jnp.int32))
    return out[None]


# speedup vs baseline: 2.5059x; 1.0021x over previous
import functools

import jax
import jax.numpy as jnp
from jax import lax
from jax.experimental import pallas as pl
from jax.experimental.pallas import tpu as pltpu
from jax.experimental.pallas import tpu_sc as plsc

_V, _D, _B = 1000000, 64, 16384


@functools.lru_cache(maxsize=None)
def _build_gather():
    info = plsc.get_sparse_core_info()
    nc, ns = info.num_cores, info.num_subcores
    nw = nc * ns
    b_per_w = _B // nw
    mesh = plsc.VectorSubcoreMesh(core_axis_name="c", subcore_axis_name="s")

    @functools.partial(
        pl.kernel,
        mesh=mesh,
        out_type=jax.ShapeDtypeStruct((_B, _D), jnp.float32),
        scratch_types=[
            pltpu.VMEM((b_per_w,), jnp.int32),
            pltpu.VMEM((b_per_w, _D), jnp.float32),
            pltpu.SemaphoreType.DMA,
            pltpu.SemaphoreType.DMA,
        ],
        compiler_params=pltpu.CompilerParams(disable_bounds_checks=True),
    )
    def gather_kernel(table_hbm, idx_hbm, out_hbm, idx_v, rows_v, sem, sem_out):
        wid = lax.axis_index("s") * nc + lax.axis_index("c")
        base = wid * b_per_w
        ngroups = b_per_w // 16

        pltpu.sync_copy(idx_hbm.at[pl.ds(base, b_per_w)], idx_v)

        def fire_group(g):
            vec = idx_v[pl.ds(g * 16, 16)]
            for j in range(16):
                r = vec[j]
                pltpu.make_async_copy(
                    table_hbm.at[pl.ds(r, 1)],
                    rows_v.at[pl.ds(g * 16 + j, 1)],
                    sem,
                ).start()

        def drain_group(g):
            pltpu.make_async_copy(
                table_hbm.at[pl.ds(0, 16)], rows_v.at[pl.ds(0, 16)], sem
            ).wait()
            pltpu.make_async_copy(
                rows_v.at[pl.ds(g * 16, 16)],
                out_hbm.at[pl.ds(base + g * 16, 16)],
                sem_out,
            ).start()

        def prime_body(g, carry):
            fire_group(g)
            return carry

        def main_body(g, carry):
            fire_group(g)
            drain_group(g - nprime)
            return carry

        def tail_body(g, carry):
            drain_group(ngroups - nprime + g)
            return carry

        def out_wait_body(g, carry):
            pltpu.make_async_copy(
                rows_v.at[pl.ds(0, 16)], out_hbm.at[pl.ds(0, 16)], sem_out
            ).wait()
            return carry

        nprime = 4
        lax.fori_loop(0, nprime, prime_body, 0)

        @plsc.parallel_loop(nprime, ngroups)
        def _(g):
            main_body(g, 0)

        lax.fori_loop(0, nprime, tail_body, 0)
        lax.fori_loop(0, ngroups, out_wait_body, 0)

    return gather_kernel


def kernel(inputs, index):
    out = _build_gather()(inputs, index.astype(jnp.int32))
    return out[None]
